# baseline (speedup 1.0000x reference)
import functools

import jax
import jax.numpy as jnp
from jax import lax
from jax.experimental import pallas as pl
from jax.experimental.pallas import tpu as pltpu
from jax.experimental.pallas import tpu_sc as plsc


def _sc_gather(table, idx):
    V, D = table.shape
    N = idx.shape[0]
    info = plsc.get_sparse_core_info()
    NC, NS = info.num_cores, info.num_subcores
    NW = NC * NS
    n_per_w = N // NW
    CHUNK = 256
    n_chunks = n_per_w // CHUNK
    mesh = plsc.VectorSubcoreMesh(core_axis_name="c", subcore_axis_name="s")

    @functools.partial(
        pl.kernel, mesh=mesh,
        out_type=jax.ShapeDtypeStruct((N, D), jnp.float32),
        scratch_types=(
            [pltpu.VMEM((CHUNK,), jnp.int32)] * n_chunks
            + [pltpu.VMEM((CHUNK, D), jnp.float32)] * 2
            + [pltpu.SemaphoreType.DMA] * 2
        ),
    )
    def k(table_hbm, idx_hbm, out_hbm, *refs):
        idxs = refs[:n_chunks]
        rows = refs[n_chunks:n_chunks + 2]
        sems = refs[n_chunks + 2:]
        wid = lax.axis_index("s") * NC + lax.axis_index("c")
        base = wid * n_per_w
        for j in range(n_chunks):
            pltpu.sync_copy(idx_hbm.at[pl.ds(base + j * CHUNK, CHUNK)], idxs[j])
        cps = [None] * n_chunks
        for j in range(min(2, n_chunks)):
            cps[j] = pltpu.async_copy(table_hbm.at[idxs[j]], rows[j % 2],
                                      sems[j % 2])
        for j in range(n_chunks):
            cps[j].wait()
            pltpu.sync_copy(rows[j % 2], out_hbm.at[pl.ds(base + j * CHUNK, CHUNK)])
            if j + 2 < n_chunks:
                cps[j + 2] = pltpu.async_copy(table_hbm.at[idxs[j + 2]],
                                              rows[j % 2], sems[j % 2])

    return k(table, idx)


T_TILE = 256
NBUF = 4


def _tc_body(*refs, dilations, row_off, has_prev):
    (x_ref, pa_in_ref, pa_w_ref, pa_b_ref, wrx_ref, wrpa_ref,
     red_b_ref, stw0_ref, stw1_ref, stb_ref, hw_ref, hb_ref,
     hpw_ref, hpb_ref) = refs[:14]
    rest = refs[14 + (1 if has_prev else 0):]
    out_ref, h_bf, h_f32, bufs, sems = rest
    T, C = x_ref.shape[1], x_ref.shape[2]
    NA = hw_ref.shape[1]
    f32 = jnp.float32
    b = pl.program_id(0)
    nb = pl.num_programs(0)

    x = x_ref[0]
    pos_amp = pa_in_ref[0]
    pa = jnp.dot(pos_amp, pa_w_ref[...], preferred_element_type=f32) + pa_b_ref[...]
    h = (jnp.dot(x, wrx_ref[...], preferred_element_type=f32)
         + jnp.dot(pa, wrpa_ref[...], preferred_element_type=f32)
         + red_b_ref[...])
    for i, d in enumerate(dilations):
        h_shift = jnp.concatenate(
            [jnp.zeros((d, C), f32), h[:T - d, :]], axis=0)
        z = (jnp.dot(h_shift, stw0_ref[i], preferred_element_type=f32)
             + jnp.dot(h, stw1_ref[i], preferred_element_type=f32)
             + stb_ref[i:i + 1, :])
        z = jnp.where(z >= 0, z, 0.2 * z)
        h = h + z
    h_f32[...] = h
    h_bf[...] = h.astype(jnp.bfloat16)

    cps = []
    for k in range(NBUF):
        row0 = k * T_TILE
        cp = pltpu.make_async_copy(
            bufs.at[k], out_ref.at[b + row_off, pl.ds(row0, T_TILE), :],
            sems.at[k])
        cps.append(cp)

        @pl.when(b > 0)
        def _drain_prev(cp=cp):
            cp.wait()

        bufs[k, :, :NA] = (
            jnp.dot(h_bf[pl.ds(row0, T_TILE), :], hw_ref[...],
                    preferred_element_type=f32) + hb_ref[...])
        bufs[k, :, NA:] = (
            jnp.dot(h_f32[pl.ds(row0, T_TILE), :], hpw_ref[...],
                    preferred_element_type=f32) + hpb_ref[...])
        cp.start()

    @pl.when(b == nb - 1)
    def _final_drain():
        for cp in cps:
            cp.wait()


def _tc_call(x, pos_amp, weights, out_shape, row_off, prev, dilations):
    nb, T, C = x.shape
    NA = out_shape.shape[2] - 2
    full = lambda shape: pl.BlockSpec(shape, lambda b: (0,) * len(shape))
    in_specs = [
        pl.BlockSpec((1, T, C), lambda b: (b, 0, 0)),
        pl.BlockSpec((1, T, 2), lambda b: (b, 0, 0)),
        full((2, C)),
        full((1, C)),
        full((C, C)),
        full((C, C)),
        full((1, C)),
        full((6, C, C)),
        full((6, C, C)),
        full((6, C)),
        full((C, NA)),
        full((1, NA)),
        full((C, 2)),
        full((1, 2)),
    ]
    operands = [x, pos_amp] + list(weights)
    kwargs = {}
    has_prev = prev is not None
    if has_prev:
        in_specs.append(pl.BlockSpec(memory_space=pl.ANY))
        operands.append(prev)
        kwargs["input_output_aliases"] = {14: 0}
    return pl.pallas_call(
        functools.partial(_tc_body, dilations=dilations, row_off=row_off,
                          has_prev=has_prev),
        grid=(nb,),
        in_specs=in_specs,
        out_specs=pl.BlockSpec(memory_space=pl.ANY),
        out_shape=out_shape,
        scratch_shapes=[
            pltpu.VMEM((T, C), jnp.bfloat16),
            pltpu.VMEM((T, C), jnp.float32),
            pltpu.VMEM((NBUF, T_TILE, NA + 2), jnp.float32),
            pltpu.SemaphoreType.DMA((NBUF,)),
        ],
        compiler_params=pltpu.CompilerParams(
            dimension_semantics=("arbitrary",)),
        **kwargs,
    )(*operands)


def kernel(atoms, pos_amp, embed_table, pa_w, pa_b, reduce_w, reduce_b,
           stack_w, stack_b, head_atom_w, head_atom_b, head_pa_w, head_pa_b):
    B, T = atoms.shape
    NA, C = embed_table.shape
    dilations = (1, 3, 9, 27, 81, 1)

    idx = atoms.reshape(-1).astype(jnp.int32)

    wrx = reduce_w[:, :C, 0].T
    wrpa = reduce_w[:, C:, 0].T
    stw0 = jnp.transpose(stack_w[..., 0], (0, 2, 1))
    stw1 = jnp.transpose(stack_w[..., 1], (0, 2, 1))
    weights = (pa_w, pa_b.reshape(1, C), wrx, wrpa, reduce_b.reshape(1, C),
               stw0, stw1, stack_b, head_atom_w.astype(jnp.bfloat16),
               head_atom_b.reshape(1, NA), head_pa_w, head_pa_b.reshape(1, 2))

    half = B // 2
    Nh = half * T
    out_shape = jax.ShapeDtypeStruct((B, T, NA + 2), jnp.float32)

    x0 = _sc_gather(embed_table, idx[:Nh]).reshape(half, T, C)
    x1 = _sc_gather(embed_table, idx[Nh:]).reshape(half, T, C)
    out0 = _tc_call(x0, pos_amp[:half], weights, out_shape, 0, None, dilations)
    out = _tc_call(x1, pos_amp[half:], weights, out_shape, half, out0, dilations)
    return out


# speedup vs baseline: 1.0150x; 1.0150x over previous
import functools

import jax
import jax.numpy as jnp
from jax import lax
from jax.experimental import pallas as pl
from jax.experimental.pallas import tpu as pltpu
from jax.experimental.pallas import tpu_sc as plsc


def _sc_gather(table, idx):
    V, D = table.shape
    N = idx.shape[0]
    info = plsc.get_sparse_core_info()
    NC, NS = info.num_cores, info.num_subcores
    NW = NC * NS
    n_per_w = N // NW
    CHUNK = 256
    n_chunks = n_per_w // CHUNK
    mesh = plsc.VectorSubcoreMesh(core_axis_name="c", subcore_axis_name="s")

    @functools.partial(
        pl.kernel, mesh=mesh,
        out_type=jax.ShapeDtypeStruct((N, D), jnp.float32),
        scratch_types=(
            [pltpu.VMEM((CHUNK,), jnp.int32)] * n_chunks
            + [pltpu.VMEM((CHUNK, D), jnp.float32)] * 2
            + [pltpu.SemaphoreType.DMA] * 2
        ),
    )
    def k(table_hbm, idx_hbm, out_hbm, *refs):
        idxs = refs[:n_chunks]
        rows = refs[n_chunks:n_chunks + 2]
        sems = refs[n_chunks + 2:]
        wid = lax.axis_index("s") * NC + lax.axis_index("c")
        base = wid * n_per_w
        for j in range(n_chunks):
            pltpu.sync_copy(idx_hbm.at[pl.ds(base + j * CHUNK, CHUNK)], idxs[j])
        cps = [None] * n_chunks
        for j in range(min(2, n_chunks)):
            cps[j] = pltpu.async_copy(table_hbm.at[idxs[j]], rows[j % 2],
                                      sems[j % 2])
        for j in range(n_chunks):
            cps[j].wait()
            pltpu.sync_copy(rows[j % 2], out_hbm.at[pl.ds(base + j * CHUNK, CHUNK)])
            if j + 2 < n_chunks:
                cps[j + 2] = pltpu.async_copy(table_hbm.at[idxs[j + 2]],
                                              rows[j % 2], sems[j % 2])

    return k(table, idx)


def _tc_body(x_ref, pa_in_ref, pa_w_ref, pa_b_ref, wrx_ref, wrpa_ref,
             red_b_ref, stw0_ref, stw1_ref, stb_ref, hw_ref, hb_ref,
             hpw_ref, hpb_ref, out_ref, *, dilations):
    T, C = x_ref.shape[1], x_ref.shape[2]
    NA = hw_ref.shape[1]
    f32 = jnp.float32
    x = x_ref[0]
    pos_amp = pa_in_ref[0]
    pa = jnp.dot(pos_amp, pa_w_ref[...], preferred_element_type=f32) + pa_b_ref[...]
    h = (jnp.dot(x, wrx_ref[...], preferred_element_type=f32)
         + jnp.dot(pa, wrpa_ref[...], preferred_element_type=f32)
         + red_b_ref[...])
    for i, d in enumerate(dilations):
        h_shift = jnp.concatenate([jnp.zeros((d, C), f32), h[:T - d, :]], axis=0)
        z = (jnp.dot(h_shift, stw0_ref[i], preferred_element_type=f32)
             + jnp.dot(h, stw1_ref[i], preferred_element_type=f32)
             + stb_ref[i:i + 1, :])
        z = jnp.where(z >= 0, z, 0.2 * z)
        h = h + z
    logits = jnp.dot(h.astype(jnp.bfloat16), hw_ref[...],
                     preferred_element_type=f32) + hb_ref[...]
    pa_out = jnp.dot(h, hpw_ref[...], preferred_element_type=f32) + hpb_ref[...]
    out_ref[0, :, :NA] = logits
    out_ref[0, :, NA:] = pa_out


def kernel(atoms, pos_amp, embed_table, pa_w, pa_b, reduce_w, reduce_b,
           stack_w, stack_b, head_atom_w, head_atom_b, head_pa_w, head_pa_b):
    B, T = atoms.shape
    NA, C = embed_table.shape
    dilations = (1, 3, 9, 27, 81, 1)

    idx = atoms.reshape(-1).astype(jnp.int32)
    x = _sc_gather(embed_table, idx).reshape(B, T, C)

    wrx = reduce_w[:, :C, 0].T
    wrpa = reduce_w[:, C:, 0].T
    stw0 = jnp.transpose(stack_w[..., 0], (0, 2, 1))
    stw1 = jnp.transpose(stack_w[..., 1], (0, 2, 1))
    pa_b2 = pa_b.reshape(1, C)
    red_b2 = reduce_b.reshape(1, C)
    hb2 = head_atom_b.reshape(1, NA)
    hpb2 = head_pa_b.reshape(1, 2)

    full = lambda shape: pl.BlockSpec(shape, lambda b: (0,) * len(shape))
    out = pl.pallas_call(
        functools.partial(_tc_body, dilations=dilations),
        grid=(B,),
        in_specs=[
            pl.BlockSpec((1, T, C), lambda b: (b, 0, 0)),
            pl.BlockSpec((1, T, 2), lambda b: (b, 0, 0)),
            full((2, C)),
            full((1, C)),
            full((C, C)),
            full((C, C)),
            full((1, C)),
            full((len(dilations), C, C)),
            full((len(dilations), C, C)),
            full((len(dilations), C)),
            full((C, NA)),
            full((1, NA)),
            full((C, 2)),
            full((1, 2)),
        ],
        out_specs=pl.BlockSpec((1, T, NA + 2), lambda b: (b, 0, 0)),
        out_shape=jax.ShapeDtypeStruct((B, T, NA + 2), jnp.float32),
        compiler_params=pltpu.CompilerParams(
            dimension_semantics=("parallel",)),
    )(x, pos_amp, pa_w, pa_b2, wrx, wrpa, red_b2, stw0, stw1, stack_b,
      head_atom_w.astype(jnp.bfloat16), hb2, head_pa_w, hpb2)
    return out
